# baseline (speedup 1.0000x reference)
import functools

import jax
import jax.numpy as jnp
from jax import lax
from jax.experimental import pallas as pl
from jax.experimental.pallas import tpu as pltpu
from jax.experimental.pallas import tpu_sc as plsc

_K = 5
_L = 16
_NC = 2
_NS = 16
_NW = _NC * _NS
_B = 1024
_N = 100000
_RW = _B // _NW
_SEG = _N // 2
_SEGC = _SEG // _L


def _chunk_body(buf):
    def body(c, carry):
        t1, t2, t3, t4, t5 = carry
        v = buf[pl.ds(c * _L, _L)]
        m1 = jnp.maximum(t1, v)
        c1 = jnp.minimum(t1, v)
        m2 = jnp.maximum(t2, c1)
        c2 = jnp.minimum(t2, c1)
        m3 = jnp.maximum(t3, c2)
        c3 = jnp.minimum(t3, c2)
        m4 = jnp.maximum(t4, c3)
        c4 = jnp.minimum(t4, c3)
        m5 = jnp.maximum(t5, c4)
        return (m1, m2, m3, m4, m5)

    return body


def _body(x_hbm, y_hbm, out_hbm, buf0, buf1, y_v, out_v, sem0, sem1):
    wid = lax.axis_index("s") * _NC + lax.axis_index("c")
    base = wid * _RW

    pltpu.sync_copy(y_hbm.at[pl.ds(base, _RW)], y_v)

    lanes = lax.iota(jnp.int32, _L)
    lane0 = lanes == 0
    neg_inf = jnp.float32(-jnp.inf)

    def process_segment(buf, seg_off, y_i, stack):
        def fix():
            off = y_i - seg_off
            c_y = off // _L
            l_y = off % _L
            vy = buf[pl.ds(c_y * _L, _L)]
            eq = lanes == l_y
            s = jnp.sum(jnp.where(eq, vy, 0.0))
            buf[pl.ds(c_y * _L, _L)] = jnp.where(eq, 0.0, vy)
            return s

        in_seg = (y_i >= seg_off) & (y_i < seg_off + _SEG)
        s_y = lax.cond(in_seg, fix, lambda: jnp.float32(0.0))
        stack = lax.fori_loop(0, _SEGC, _chunk_body(buf), stack, unroll=8)
        return stack, s_y

    def row_loop(j, loss_acc):
        row = base + j
        pltpu.async_copy(x_hbm.at[row, pl.ds(_SEG, _SEG)], buf1, sem1)

        y_vec = y_v[pl.ds((j // _L) * _L, _L)]
        y_i = jnp.sum(jnp.where(lanes == (j % _L), y_vec, 0))

        init = tuple(jnp.full((_L,), neg_inf) for _ in range(_K))
        pltpu.make_async_copy(x_hbm.at[row, pl.ds(0, _SEG)], buf0,
                              sem0).wait()
        stack, sy0 = process_segment(buf0, 0, y_i, init)

        @pl.when(j + 1 < _RW)
        def _prefetch():
            pltpu.async_copy(x_hbm.at[row + 1, pl.ds(0, _SEG)], buf0, sem0)

        pltpu.make_async_copy(x_hbm.at[row, pl.ds(_SEG, _SEG)], buf1,
                              sem1).wait()
        (t1, t2, t3, t4, t5), sy1 = process_segment(buf1, _SEG, y_i, stack)
        s_y = sy0 + sy1

        for _ in range(_K - 1):
            m = jnp.max(t1)
            ffs = plsc.all_reduce_ffs(t1 == m)
            sel = lanes == ffs
            t1 = jnp.where(sel, t2, t1)
            t2 = jnp.where(sel, t3, t2)
            t3 = jnp.where(sel, t4, t3)
            t4 = jnp.where(sel, t5, t4)
            t5 = jnp.where(sel, neg_inf, t5)
        s_topk = jnp.max(t1)

        hinge = jnp.maximum(1.0 + s_topk - s_y, 0.0)
        return loss_acc + jnp.where(lane0, hinge, 0.0)

    pltpu.async_copy(x_hbm.at[base, pl.ds(0, _SEG)], buf0, sem0)
    loss_acc = lax.fori_loop(0, _RW, row_loop, jnp.zeros((_L,), jnp.float32))

    out_v[...] = loss_acc
    pltpu.sync_copy(out_v, out_hbm.at[wid])


@jax.jit
def kernel(x, y):
    mesh = plsc.VectorSubcoreMesh(core_axis_name="c", subcore_axis_name="s")
    partials = pl.kernel(
        _body,
        out_type=jax.ShapeDtypeStruct((_NW, _L), jnp.float32),
        mesh=mesh,
        compiler_params=pltpu.CompilerParams(needs_layout_passes=False,
                                             use_tc_tiling_on_sc=False),
        scratch_types=[
            pltpu.VMEM((_SEG,), jnp.float32),
            pltpu.VMEM((_SEG,), jnp.float32),
            pltpu.VMEM((_RW,), jnp.int32),
            pltpu.VMEM((_L,), jnp.float32),
            pltpu.SemaphoreType.DMA,
            pltpu.SemaphoreType.DMA,
        ],
    )(x, y)
    return jnp.sum(partials[:, 0]) / jnp.float32(_B)


# speedup vs baseline: 1.7019x; 1.7019x over previous
import functools

import jax
import jax.numpy as jnp
from jax import lax
from jax.experimental import pallas as pl
from jax.experimental.pallas import tpu as pltpu
from jax.experimental.pallas import tpu_sc as plsc

_K = 5
_L = 16
_NC = 2
_NS = 16
_NW = _NC * _NS
_B = 1024
_N = 100000
_RW = _B // _NW
_SEG0 = 49920
_SEG1 = _N - _SEG0
_SEGC0 = _SEG0 // _L
_SEGC1 = _SEG1 // _L


def _chunk_body(buf):
    def body(c, carry):
        t1, t2, t3, t4, t5 = carry
        v = buf[pl.ds(c * _L, _L)]
        m1 = jnp.maximum(t1, v)
        c1 = jnp.minimum(t1, v)
        m2 = jnp.maximum(t2, c1)
        c2 = jnp.minimum(t2, c1)
        m3 = jnp.maximum(t3, c2)
        c3 = jnp.minimum(t3, c2)
        m4 = jnp.maximum(t4, c3)
        c4 = jnp.minimum(t4, c3)
        m5 = jnp.maximum(t5, c4)
        return (m1, m2, m3, m4, m5)

    return body


def _body(x_hbm, y_hbm, out_hbm, buf0, buf1, y_v, out_v, sem0, sem1):
    wid = lax.axis_index("s") * _NC + lax.axis_index("c")
    base = wid * _RW

    pltpu.sync_copy(y_hbm.at[pl.ds(base, _RW)], y_v)

    lanes = lax.iota(jnp.int32, _L)
    lane0 = lanes == 0
    neg_inf = jnp.float32(-jnp.inf)

    def process_segment(buf, seg_off, seg_len, n_chunks, y_i, stack):
        def fix():
            off = y_i - seg_off
            c_y = off // _L
            l_y = off % _L
            vy = buf[pl.ds(c_y * _L, _L)]
            eq = lanes == l_y
            s = jnp.sum(jnp.where(eq, vy, 0.0))
            buf[pl.ds(c_y * _L, _L)] = jnp.where(eq, 0.0, vy)
            return s

        in_seg = (y_i >= seg_off) & (y_i < seg_off + seg_len)
        s_y = lax.cond(in_seg, fix, lambda: jnp.float32(0.0))
        stack = lax.fori_loop(0, n_chunks, _chunk_body(buf), stack, unroll=8)
        return stack, s_y

    def row_loop(j, loss_acc):
        row = base + j
        pltpu.async_copy(x_hbm.at[row, pl.ds(_SEG0, _SEG1)], buf1, sem1)

        y_vec = y_v[pl.ds((j // _L) * _L, _L)]
        y_i = jnp.sum(jnp.where(lanes == (j % _L), y_vec, 0))

        init = tuple(jnp.full((_L,), neg_inf) for _ in range(_K))
        pltpu.make_async_copy(x_hbm.at[row, pl.ds(0, _SEG0)], buf0,
                              sem0).wait()
        stack, sy0 = process_segment(buf0, 0, _SEG0, _SEGC0, y_i, init)

        @pl.when(j + 1 < _RW)
        def _prefetch():
            pltpu.async_copy(x_hbm.at[row + 1, pl.ds(0, _SEG0)], buf0, sem0)

        pltpu.make_async_copy(x_hbm.at[row, pl.ds(_SEG0, _SEG1)], buf1,
                              sem1).wait()
        (t1, t2, t3, t4, t5), sy1 = process_segment(buf1, _SEG0, _SEG1,
                                                    _SEGC1, y_i, stack)
        s_y = sy0 + sy1

        for _ in range(_K - 1):
            m = jnp.max(t1)
            ffs = plsc.all_reduce_ffs(t1 == m)
            sel = lanes == ffs
            t1 = jnp.where(sel, t2, t1)
            t2 = jnp.where(sel, t3, t2)
            t3 = jnp.where(sel, t4, t3)
            t4 = jnp.where(sel, t5, t4)
            t5 = jnp.where(sel, neg_inf, t5)
        s_topk = jnp.max(t1)

        hinge = jnp.maximum(1.0 + s_topk - s_y, 0.0)
        return loss_acc + jnp.where(lane0, hinge, 0.0)

    pltpu.async_copy(x_hbm.at[base, pl.ds(0, _SEG0)], buf0, sem0)
    loss_acc = lax.fori_loop(0, _RW, row_loop, jnp.zeros((_L,), jnp.float32))

    out_v[...] = loss_acc
    pltpu.sync_copy(out_v, out_hbm.at[wid])


@jax.jit
def kernel(x, y):
    mesh = plsc.VectorSubcoreMesh(core_axis_name="c", subcore_axis_name="s")
    partials = pl.kernel(
        _body,
        out_type=jax.ShapeDtypeStruct((_NW, _L), jnp.float32),
        mesh=mesh,
        compiler_params=pltpu.CompilerParams(needs_layout_passes=False),
        scratch_types=[
            pltpu.VMEM((_SEG0,), jnp.float32),
            pltpu.VMEM((_SEG1,), jnp.float32),
            pltpu.VMEM((_RW,), jnp.int32),
            pltpu.VMEM((_L,), jnp.float32),
            pltpu.SemaphoreType.DMA,
            pltpu.SemaphoreType.DMA,
        ],
    )(x, y)
    return jnp.sum(partials[:, 0]) / jnp.float32(_B)
